# baseline (speedup 1.0000x reference)
import functools

import jax
import jax.numpy as jnp
from jax import lax
from jax.experimental import pallas as pl
from jax.experimental.pallas import tpu as pltpu
from jax.experimental.pallas import tpu_sc as plsc

B = 4096
D0 = 2048
D1 = 1024
D2 = 512
LATENT = 2048
NUM_MODES = 64
MODE_DIM = 512

TB = 256
JB = 512
NB = 18
NSLOT = NB * TB


def _tables_body(mt_ref, ws_ref, bs_ref, wh_ref, bh_ref, st_ref, ht_ref):
    mt = mt_ref[...]
    dn = (((1,), (1,)), ((), ()))
    st_ref[...] = jax.lax.dot_general(
        mt, ws_ref[...], dn, preferred_element_type=jnp.float32) + bs_ref[...]
    ht_ref[...] = jax.lax.dot_general(
        mt, wh_ref[...], dn, preferred_element_type=jnp.float32) + bh_ref[...]


def _film_tables(mode_table, Ws, bs, Wh, bh):
    return pl.pallas_call(
        _tables_body,
        out_shape=(
            jax.ShapeDtypeStruct((NUM_MODES, LATENT), jnp.float32),
            jax.ShapeDtypeStruct((NUM_MODES, LATENT), jnp.float32),
        ),
    )(mode_table, Ws, bs.reshape(1, LATENT), Wh, bh.reshape(1, LATENT))


@functools.lru_cache(maxsize=None)
def _make_sc_row_gather(n_out, d, dtype, rows_per_chunk):
    info = plsc.get_sparse_core_info()
    nw = info.num_cores * info.num_subcores
    per_w = n_out // nw
    n_chunks = per_w // rows_per_chunk
    assert per_w % rows_per_chunk == 0 and rows_per_chunk % 8 == 0
    mesh = plsc.VectorSubcoreMesh(core_axis_name="c", subcore_axis_name="s")

    @functools.partial(
        pl.kernel, mesh=mesh,
        out_type=jax.ShapeDtypeStruct((n_out, d), dtype),
        scratch_types=[
            pltpu.VMEM((per_w,), jnp.int32),
            pltpu.VMEM((rows_per_chunk, d), dtype),
            pltpu.SemaphoreType.DMA,
        ],
    )
    def gather_k(table_hbm, idx_hbm, out_hbm, idx_v, rows_v, sem):
        wid = lax.axis_index("s") * info.num_cores + lax.axis_index("c")
        base = wid * per_w
        pltpu.sync_copy(idx_hbm.at[pl.ds(base, per_w)], idx_v)
        for c in range(n_chunks):
            off = c * rows_per_chunk
            pltpu.async_copy(
                table_hbm.at[idx_v.at[pl.ds(off, rows_per_chunk)]],
                rows_v, sem).wait()
            pltpu.sync_copy(rows_v, out_hbm.at[pl.ds(base + off, rows_per_chunk)])

    return gather_k


def _gather_x(table, idx):
    return _make_sc_row_gather(NSLOT, D0, jnp.float32, 48)(table, idx)


def _gather_out(table, idx):
    return _make_sc_row_gather(B, LATENT, jnp.float32, 32)(table, idx)


def _mm_body(be_ref, x_ref, w0_ref, b0_ref, w1_ref, b1_ref, w2_ref, b2_ref,
             st_ref, ht_ref, mode_ref, out_ref):
    i = pl.program_id(0)
    j = pl.program_id(1)
    e = be_ref[i]
    jsl = pl.ds(j * JB, JB)
    x = x_ref[...].astype(jnp.bfloat16)
    mode = mode_ref[0, 0, :]
    iota = jax.lax.broadcasted_iota(jnp.int32, (TB, NUM_MODES), 1)
    onehot = (mode[:, None] == iota).astype(jnp.bfloat16)
    dng = (((1,), (0,)), ((), ()))
    s = jax.lax.dot_general(onehot, st_ref[:, jsl], dng,
                            preferred_element_type=jnp.float32)
    h = jax.lax.dot_general(onehot, ht_ref[:, jsl], dng,
                            preferred_element_type=jnp.float32)
    dn = (((1,), (1,)), ((), ()))

    @pl.when(e == 0)
    def _():
        p = jax.lax.dot_general(x, w0_ref[jsl, :], dn,
                                preferred_element_type=jnp.float32)
        p = p + b0_ref[:, jsl]
        out_ref[...] = p * (1.0 + s) + h

    @pl.when(e == 1)
    def _():
        p = jax.lax.dot_general(x[:, :D1], w1_ref[jsl, :], dn,
                                preferred_element_type=jnp.float32)
        p = p + b1_ref[:, jsl]
        out_ref[...] = p * (1.0 + s) + h

    @pl.when(e == 2)
    def _():
        p = jax.lax.dot_general(x[:, :D2], w2_ref[jsl, :], dn,
                                preferred_element_type=jnp.float32)
        p = p + b2_ref[:, jsl]
        out_ref[...] = p * (1.0 + s) + h


def _grouped_mm(block_expert, x_sorted, w0, b0, w1, b1, w2, b2, stb, htb, mode3):
    grid_spec = pltpu.PrefetchScalarGridSpec(
        num_scalar_prefetch=1,
        grid=(NB, LATENT // JB),
        in_specs=[
            pl.BlockSpec((TB, D0), lambda i, j, be: (i, 0)),
            pl.BlockSpec((LATENT, D0), lambda i, j, be: (0, 0)),
            pl.BlockSpec((1, LATENT), lambda i, j, be: (0, 0)),
            pl.BlockSpec((LATENT, D1), lambda i, j, be: (0, 0)),
            pl.BlockSpec((1, LATENT), lambda i, j, be: (0, 0)),
            pl.BlockSpec((LATENT, D2), lambda i, j, be: (0, 0)),
            pl.BlockSpec((1, LATENT), lambda i, j, be: (0, 0)),
            pl.BlockSpec((NUM_MODES, LATENT), lambda i, j, be: (0, 0)),
            pl.BlockSpec((NUM_MODES, LATENT), lambda i, j, be: (0, 0)),
            pl.BlockSpec((1, 1, TB), lambda i, j, be: (i, 0, 0)),
        ],
        out_specs=pl.BlockSpec((TB, JB), lambda i, j, be: (i, j)),
    )
    return pl.pallas_call(
        _mm_body,
        grid_spec=grid_spec,
        out_shape=jax.ShapeDtypeStruct((NSLOT, LATENT), jnp.float32),
    )(block_expert, x_sorted, w0, b0.reshape(1, LATENT), w1,
      b1.reshape(1, LATENT), w2, b2.reshape(1, LATENT), stb, htb, mode3)


@jax.jit
def kernel(features, modality_ids, mode_ids, W0, b0, W1, b1, W2, b2,
           mode_table, Ws, bs, Wh, bh):
    mids = modality_ids.astype(jnp.int32)
    oh = (mids[:, None] == jnp.arange(3, dtype=jnp.int32)[None, :]).astype(jnp.int32)
    counts = oh.sum(axis=0)
    pc = ((counts + TB - 1) // TB) * TB
    start = jnp.concatenate(
        [jnp.zeros((1,), jnp.int32), jnp.cumsum(pc)[:2].astype(jnp.int32)])
    rank = jnp.cumsum(oh, axis=0) - oh
    rank_t = jnp.take_along_axis(rank, mids[:, None], axis=1)[:, 0]
    slots = start[mids] + rank_t.astype(jnp.int32)
    ridx = jnp.zeros((NSLOT,), jnp.int32).at[slots].set(
        jnp.arange(B, dtype=jnp.int32))
    bstart = jnp.arange(NB, dtype=jnp.int32) * TB
    block_expert = ((bstart >= pc[0]).astype(jnp.int32)
                    + (bstart >= pc[0] + pc[1]).astype(jnp.int32))
    mode_sorted = jnp.take(mode_ids.astype(jnp.int32), ridx)
    mode3 = mode_sorted.reshape(NB, 1, TB)

    st, ht = _film_tables(mode_table, Ws, bs, Wh, bh)
    stb = st.astype(jnp.bfloat16)
    htb = ht.astype(jnp.bfloat16)
    w0 = W0.astype(jnp.bfloat16)
    w1 = W1.astype(jnp.bfloat16)
    w2 = W2.astype(jnp.bfloat16)

    x_sorted = _gather_x(features, ridx)
    y_sorted = _grouped_mm(block_expert, x_sorted, w0, b0, w1, b1, w2, b2,
                           stb, htb, mode3)
    out = _gather_out(y_sorted, slots)
    return out


# speedup vs baseline: 2.5361x; 2.5361x over previous
import jax
import jax.numpy as jnp
from jax.experimental import pallas as pl
from jax.experimental.pallas import tpu as pltpu

B = 4096
D0 = 2048
D1 = 1024
D2 = 512
LATENT = 2048
NUM_MODES = 64
MODE_DIM = 512

TB = 256


def _tables_body(mt_ref, ws_ref, bs_ref, wh_ref, bh_ref, st_ref, ht_ref):
    mt = mt_ref[...]
    dn = (((1,), (1,)), ((), ()))
    st_ref[...] = jax.lax.dot_general(
        mt, ws_ref[...], dn, preferred_element_type=jnp.float32) + bs_ref[...]
    ht_ref[...] = jax.lax.dot_general(
        mt, wh_ref[...], dn, preferred_element_type=jnp.float32) + bh_ref[...]


def _film_tables(mode_table, Ws, bs, Wh, bh):
    return pl.pallas_call(
        _tables_body,
        out_shape=(
            jax.ShapeDtypeStruct((NUM_MODES, LATENT), jnp.float32),
            jax.ShapeDtypeStruct((NUM_MODES, LATENT), jnp.float32),
        ),
    )(mode_table, Ws, bs.reshape(1, LATENT), Wh, bh.reshape(1, LATENT))


def _main_body(x_ref, mod_ref, mode_ref, w0_ref, w1_ref, w2_ref,
               b0_ref, b1_ref, b2_ref, st_ref, ht_ref, out_ref,
               w0b, w1b, w2b, stb, htb):
    i = pl.program_id(0)

    @pl.when(i == 0)
    def _():
        w0b[...] = w0_ref[...].astype(jnp.bfloat16)
        w1b[...] = w1_ref[...].astype(jnp.bfloat16)
        w2b[...] = w2_ref[...].astype(jnp.bfloat16)
        stb[...] = st_ref[...].astype(jnp.bfloat16)
        htb[...] = ht_ref[...].astype(jnp.bfloat16)

    x = x_ref[...].astype(jnp.bfloat16)
    dn = (((1,), (1,)), ((), ()))
    p0 = jax.lax.dot_general(x, w0b[...], dn,
                             preferred_element_type=jnp.float32) + b0_ref[...]
    p1 = jax.lax.dot_general(x[:, :D1], w1b[...], dn,
                             preferred_element_type=jnp.float32) + b1_ref[...]
    p2 = jax.lax.dot_general(x[:, :D2], w2b[...], dn,
                             preferred_element_type=jnp.float32) + b2_ref[...]
    mod = mod_ref[0, 0, :]
    modc = mod[:, None]
    content = jnp.where(modc == 0, p0, jnp.where(modc == 1, p1, p2))

    mode = mode_ref[0, 0, :]
    iota = jax.lax.broadcasted_iota(jnp.int32, (TB, NUM_MODES), 1)
    onehot = (mode[:, None] == iota).astype(jnp.bfloat16)
    dng = (((1,), (0,)), ((), ()))
    s = jax.lax.dot_general(onehot, stb[...], dng,
                            preferred_element_type=jnp.float32)
    h = jax.lax.dot_general(onehot, htb[...], dng,
                            preferred_element_type=jnp.float32)
    out_ref[...] = content * (1.0 + s) + h


@jax.jit
def kernel(features, modality_ids, mode_ids, W0, b0, W1, b1, W2, b2,
           mode_table, Ws, bs, Wh, bh):
    st, ht = _film_tables(mode_table, Ws, bs, Wh, bh)
    mod3 = modality_ids.astype(jnp.int32).reshape(B // TB, 1, TB)
    mode3 = mode_ids.astype(jnp.int32).reshape(B // TB, 1, TB)

    out = pl.pallas_call(
        _main_body,
        grid=(B // TB,),
        in_specs=[
            pl.BlockSpec((TB, D0), lambda i: (i, 0)),
            pl.BlockSpec((1, 1, TB), lambda i: (i, 0, 0)),
            pl.BlockSpec((1, 1, TB), lambda i: (i, 0, 0)),
            pl.BlockSpec((LATENT, D0), lambda i: (0, 0)),
            pl.BlockSpec((LATENT, D1), lambda i: (0, 0)),
            pl.BlockSpec((LATENT, D2), lambda i: (0, 0)),
            pl.BlockSpec((1, LATENT), lambda i: (0, 0)),
            pl.BlockSpec((1, LATENT), lambda i: (0, 0)),
            pl.BlockSpec((1, LATENT), lambda i: (0, 0)),
            pl.BlockSpec((NUM_MODES, LATENT), lambda i: (0, 0)),
            pl.BlockSpec((NUM_MODES, LATENT), lambda i: (0, 0)),
        ],
        out_specs=pl.BlockSpec((TB, LATENT), lambda i: (i, 0)),
        out_shape=jax.ShapeDtypeStruct((B, LATENT), jnp.float32),
        scratch_shapes=[
            pltpu.VMEM((LATENT, D0), jnp.bfloat16),
            pltpu.VMEM((LATENT, D1), jnp.bfloat16),
            pltpu.VMEM((LATENT, D2), jnp.bfloat16),
            pltpu.VMEM((NUM_MODES, LATENT), jnp.bfloat16),
            pltpu.VMEM((NUM_MODES, LATENT), jnp.bfloat16),
        ],
    )(features, mod3, mode3, W0, W1, W2, b0.reshape(1, LATENT),
      b1.reshape(1, LATENT), b2.reshape(1, LATENT), st, ht)
    return out
